# Initial kernel scaffold; baseline (speedup 1.0000x reference)
import jax
import jax.numpy as jnp
from jax.experimental import pallas as pl


def kernel(x, edge_index, W_enc, b_enc, Ws0, Wn0, bb0, Ws1, Wn1, bb1, Ws2, Wn2, bb2, Wp1, bp1, Wp2, bp2):
    raise NotImplementedError("write your pallas kernel here")


# speedup vs baseline: 3.3388x; 3.3388x over previous
import jax
import jax.numpy as jnp
from jax import lax
from jax.experimental import pallas as pl
from jax.experimental.pallas import tpu as pltpu
from jax.experimental.pallas import tpu_sc as plsc

N = 10000
D = 128
H = 128
OUT = 256
E = 320000

NC = 2
NS = 16
NW = NC * NS
EW = E // NW
C = 128
NCH = 80
EP = NCH * C
TRASH = N
AGG_ROWS = N + 8
STR = 624
TAIL = N - NS * STR
DEGW = 16

ROWS_BLK = 1000
GRID = N // ROWS_BLK

_mesh = plsc.VectorSubcoreMesh(core_axis_name="c", subcore_axis_name="s",
                               num_cores=NC, num_subcores=NS)


def _zero_vec():
    return jnp.zeros((16,), jnp.float32)


def _sc_deg_body(dst_hbm, deg_out, idx_d, ones_v, deg_sh):
    cid = lax.axis_index("c")
    sid = lax.axis_index("s")
    wid = cid * NS + sid
    base = sid * STR

    def zrow(r, c):
        for k in range(H // 16):
            ones_v[r, pl.ds(k * 16, 16)] = _zero_vec()
        return c

    lax.fori_loop(0, C, zrow, 0)

    for j in range(STR // C):
        pltpu.sync_copy(ones_v, deg_sh.at[pl.ds(base + j * C, C)])
    rem = STR - (STR // C) * C
    if rem:
        pltpu.sync_copy(ones_v.at[pl.ds(0, rem)],
                        deg_sh.at[pl.ds(base + STR - rem, rem)])

    @pl.when(sid == NS - 1)
    def _():
        pltpu.sync_copy(ones_v.at[pl.ds(0, TAIL)],
                        deg_sh.at[pl.ds(NS * STR, TAIL)])

    def onesrow(r, c):
        for k in range(H // 16):
            ones_v[r, pl.ds(k * 16, 16)] = jnp.ones((16,), jnp.float32)
        return c

    lax.fori_loop(0, C, onesrow, 0)

    plsc.subcore_barrier()

    def step(j, c):
        pltpu.sync_copy(dst_hbm.at[wid, j], idx_d)
        pltpu.sync_copy(ones_v, deg_sh.at[idx_d], add=True)
        return c

    lax.fori_loop(0, NCH, step, 0)

    plsc.subcore_barrier()

    pltpu.sync_copy(deg_sh.at[pl.ds(base, STR)],
                    deg_out.at[cid, pl.ds(base, STR)])

    @pl.when(sid == NS - 1)
    def _():
        pltpu.sync_copy(deg_sh.at[pl.ds(NS * STR, TAIL)],
                        deg_out.at[cid, pl.ds(NS * STR, TAIL)])


_sc_deg = pl.kernel(
    _sc_deg_body,
    out_type=jax.ShapeDtypeStruct((NC, N, H), jnp.float32),
    mesh=_mesh,
    scratch_types=[
        pltpu.VMEM((C,), jnp.int32),
        pltpu.VMEM((C, H), jnp.float32),
        pltpu.VMEM_SHARED((AGG_ROWS, H), jnp.float32),
    ],
)


def _sc_agg_body(h_hbm, src_hbm, dst_hbm, agg_out,
                 idx_sa, idx_da, idx_sb, idx_db, rows_a, rows_b,
                 agg_sh, sem_a, sem_b):
    cid = lax.axis_index("c")
    sid = lax.axis_index("s")
    wid = cid * NS + sid
    base = sid * STR

    def zrow(r, c):
        for k in range(H // 16):
            rows_a[r, pl.ds(k * 16, 16)] = _zero_vec()
        return c

    lax.fori_loop(0, C, zrow, 0)

    for j in range(STR // C):
        pltpu.sync_copy(rows_a, agg_sh.at[pl.ds(base + j * C, C)])
    rem = STR - (STR // C) * C
    if rem:
        pltpu.sync_copy(rows_a.at[pl.ds(0, rem)],
                        agg_sh.at[pl.ds(base + STR - rem, rem)])

    @pl.when(sid == NS - 1)
    def _():
        pltpu.sync_copy(rows_a.at[pl.ds(0, TAIL)],
                        agg_sh.at[pl.ds(NS * STR, TAIL)])

    plsc.subcore_barrier()

    pltpu.sync_copy(src_hbm.at[wid, 0], idx_sa)
    pltpu.sync_copy(dst_hbm.at[wid, 0], idx_da)
    pltpu.async_copy(h_hbm.at[idx_sa], rows_a, sem_a)

    def pair(jp, c):
        j0 = jp * 2
        pltpu.make_async_copy(h_hbm.at[idx_sa], rows_a, sem_a).wait()
        pltpu.sync_copy(src_hbm.at[wid, j0 + 1], idx_sb)
        pltpu.sync_copy(dst_hbm.at[wid, j0 + 1], idx_db)
        pltpu.async_copy(h_hbm.at[idx_sb], rows_b, sem_b)
        pltpu.sync_copy(rows_a, agg_sh.at[idx_da], add=True)
        pltpu.make_async_copy(h_hbm.at[idx_sb], rows_b, sem_b).wait()

        @pl.when(jp < NCH // 2 - 1)
        def _():
            pltpu.sync_copy(src_hbm.at[wid, j0 + 2], idx_sa)
            pltpu.sync_copy(dst_hbm.at[wid, j0 + 2], idx_da)
            pltpu.async_copy(h_hbm.at[idx_sa], rows_a, sem_a)

        pltpu.sync_copy(rows_b, agg_sh.at[idx_db], add=True)
        return c

    lax.fori_loop(0, NCH // 2, pair, 0)

    plsc.subcore_barrier()

    pltpu.sync_copy(agg_sh.at[pl.ds(base, STR)],
                    agg_out.at[cid, pl.ds(base, STR)])

    @pl.when(sid == NS - 1)
    def _():
        pltpu.sync_copy(agg_sh.at[pl.ds(NS * STR, TAIL)],
                        agg_out.at[cid, pl.ds(NS * STR, TAIL)])


_sc_agg = pl.kernel(
    _sc_agg_body,
    out_type=jax.ShapeDtypeStruct((NC, N, H), jnp.float32),
    mesh=_mesh,
    scratch_types=[
        pltpu.VMEM((C,), jnp.int32),
        pltpu.VMEM((C,), jnp.int32),
        pltpu.VMEM((C,), jnp.int32),
        pltpu.VMEM((C,), jnp.int32),
        pltpu.VMEM((C, H), jnp.float32),
        pltpu.VMEM((C, H), jnp.float32),
        pltpu.VMEM_SHARED((AGG_ROWS, H), jnp.float32),
        pltpu.SemaphoreType.DMA,
        pltpu.SemaphoreType.DMA,
    ],
)


def _encode_body(x_ref, w_ref, b_ref, d_ref, o_ref, dinv_ref):
    o_ref[...] = jnp.maximum(
        jnp.dot(x_ref[...], w_ref[...], preferred_element_type=jnp.float32)
        + b_ref[...], 0.0)
    dinv = 1.0 / jnp.maximum(d_ref[0, :, 0:1] + d_ref[1, :, 0:1], 1.0)
    dinv_ref[...] = jnp.broadcast_to(dinv, dinv_ref.shape)


_encode = pl.pallas_call(
    _encode_body,
    grid=(GRID,),
    in_specs=[
        pl.BlockSpec((ROWS_BLK, D), lambda i: (i, 0)),
        pl.BlockSpec((D, H), lambda i: (0, 0)),
        pl.BlockSpec((1, H), lambda i: (0, 0)),
        pl.BlockSpec((NC, ROWS_BLK, H), lambda i: (0, i, 0)),
    ],
    out_specs=[
        pl.BlockSpec((ROWS_BLK, H), lambda i: (i, 0)),
        pl.BlockSpec((ROWS_BLK, DEGW), lambda i: (i, 0)),
    ],
    out_shape=[
        jax.ShapeDtypeStruct((N, H), jnp.float32),
        jax.ShapeDtypeStruct((N, DEGW), jnp.float32),
    ],
)


def _update_body(h_ref, a_ref, d_ref, ws_ref, wn_ref, b_ref, o_ref):
    agg = (a_ref[0] + a_ref[1]) * d_ref[:, 0:1]
    o_ref[...] = jnp.maximum(
        jnp.dot(h_ref[...], ws_ref[...], preferred_element_type=jnp.float32)
        + jnp.dot(agg, wn_ref[...], preferred_element_type=jnp.float32)
        + b_ref[...], 0.0)


_update = pl.pallas_call(
    _update_body,
    grid=(GRID,),
    in_specs=[
        pl.BlockSpec((ROWS_BLK, H), lambda i: (i, 0)),
        pl.BlockSpec((NC, ROWS_BLK, H), lambda i: (0, i, 0)),
        pl.BlockSpec((ROWS_BLK, DEGW), lambda i: (i, 0)),
        pl.BlockSpec((H, H), lambda i: (0, 0)),
        pl.BlockSpec((H, H), lambda i: (0, 0)),
        pl.BlockSpec((1, H), lambda i: (0, 0)),
    ],
    out_specs=pl.BlockSpec((ROWS_BLK, H), lambda i: (i, 0)),
    out_shape=jax.ShapeDtypeStruct((N, H), jnp.float32),
)


def _final_body(h_ref, a_ref, d_ref, ws_ref, wn_ref, b_ref,
                wp1_ref, bp1_ref, wp2_ref, bp2_ref, o_ref, acc_ref):
    i = pl.program_id(0)
    agg = (a_ref[0] + a_ref[1]) * d_ref[:, 0:1]
    h3 = jnp.maximum(
        jnp.dot(h_ref[...], ws_ref[...], preferred_element_type=jnp.float32)
        + jnp.dot(agg, wn_ref[...], preferred_element_type=jnp.float32)
        + b_ref[...], 0.0)

    @pl.when(i == 0)
    def _():
        acc_ref[...] = jnp.zeros_like(acc_ref)

    acc_ref[...] += jnp.sum(h3, axis=0, keepdims=True)

    @pl.when(i == GRID - 1)
    def _():
        g = acc_ref[...] * (1.0 / N)
        p = jnp.maximum(
            jnp.dot(g, wp1_ref[...], preferred_element_type=jnp.float32)
            + bp1_ref[...], 0.0)
        o_ref[...] = (jnp.dot(p, wp2_ref[...],
                              preferred_element_type=jnp.float32)
                      + bp2_ref[...])


_final = pl.pallas_call(
    _final_body,
    grid=(GRID,),
    in_specs=[
        pl.BlockSpec((ROWS_BLK, H), lambda i: (i, 0)),
        pl.BlockSpec((NC, ROWS_BLK, H), lambda i: (0, i, 0)),
        pl.BlockSpec((ROWS_BLK, DEGW), lambda i: (i, 0)),
        pl.BlockSpec((H, H), lambda i: (0, 0)),
        pl.BlockSpec((H, H), lambda i: (0, 0)),
        pl.BlockSpec((1, H), lambda i: (0, 0)),
        pl.BlockSpec((H, OUT), lambda i: (0, 0)),
        pl.BlockSpec((1, OUT), lambda i: (0, 0)),
        pl.BlockSpec((OUT, OUT), lambda i: (0, 0)),
        pl.BlockSpec((1, OUT), lambda i: (0, 0)),
    ],
    out_specs=pl.BlockSpec((1, OUT), lambda i: (0, 0)),
    out_shape=jax.ShapeDtypeStruct((1, OUT), jnp.float32),
    scratch_shapes=[pltpu.VMEM((1, H), jnp.float32)],
)


def kernel(x, edge_index, W_enc, b_enc, Ws0, Wn0, bb0, Ws1, Wn1, bb1,
           Ws2, Wn2, bb2, Wp1, bp1, Wp2, bp2):
    pad = EP - EW
    src = jnp.pad(edge_index[0].reshape(NW, EW), ((0, 0), (0, pad)),
                  constant_values=0).reshape(NW, NCH, C)
    dst = jnp.pad(edge_index[1].reshape(NW, EW), ((0, 0), (0, pad)),
                  constant_values=TRASH).reshape(NW, NCH, C)

    deg2 = _sc_deg(dst)
    h, dinv = _encode(x, W_enc, b_enc.reshape(1, H), deg2)
    agg = _sc_agg(h, src, dst)
    h = _update(h, agg, dinv, Ws0, Wn0, bb0.reshape(1, H))
    agg = _sc_agg(h, src, dst)
    h = _update(h, agg, dinv, Ws1, Wn1, bb1.reshape(1, H))
    agg = _sc_agg(h, src, dst)
    out = _final(h, agg, dinv, Ws2, Wn2, bb2.reshape(1, H),
                 Wp1, bp1.reshape(1, OUT), Wp2, bp2.reshape(1, OUT))
    return out.reshape(OUT)


# speedup vs baseline: 3.7741x; 1.1304x over previous
import jax
import jax.numpy as jnp
from jax import lax
from jax.experimental import pallas as pl
from jax.experimental.pallas import tpu as pltpu
from jax.experimental.pallas import tpu_sc as plsc

N = 10000
D = 128
H = 128
OUT = 256
E = 320000

NC = 2
NS = 16
NW = NC * NS
EW = E // NW
C = 128
NCH = 80
EP = NCH * C
TRASH = N
AGG_ROWS = N + 8
STR = 624
TAIL = N - NS * STR
DEGW = 16

ROWS_BLK = 1000
GRID = N // ROWS_BLK

_mesh = plsc.VectorSubcoreMesh(core_axis_name="c", subcore_axis_name="s",
                               num_cores=NC, num_subcores=NS)


def _zero_vec():
    return jnp.zeros((16,), jnp.float32)


def _sc_deg_body(dst_hbm, deg_out, idx_d, ones_v, deg_sh):
    cid = lax.axis_index("c")
    sid = lax.axis_index("s")
    wid = cid * NS + sid
    base = sid * STR

    def zrow(r, c):
        for k in range(H // 16):
            ones_v[r, pl.ds(k * 16, 16)] = _zero_vec()
        return c

    lax.fori_loop(0, C, zrow, 0)

    for j in range(STR // C):
        pltpu.sync_copy(ones_v, deg_sh.at[pl.ds(base + j * C, C)])
    rem = STR - (STR // C) * C
    if rem:
        pltpu.sync_copy(ones_v.at[pl.ds(0, rem)],
                        deg_sh.at[pl.ds(base + STR - rem, rem)])

    @pl.when(sid == NS - 1)
    def _():
        pltpu.sync_copy(ones_v.at[pl.ds(0, TAIL)],
                        deg_sh.at[pl.ds(NS * STR, TAIL)])

    def onesrow(r, c):
        for k in range(H // 16):
            ones_v[r, pl.ds(k * 16, 16)] = jnp.ones((16,), jnp.float32)
        return c

    lax.fori_loop(0, C, onesrow, 0)

    plsc.subcore_barrier()

    def step(j, c):
        pltpu.sync_copy(dst_hbm.at[wid, j], idx_d)
        pltpu.sync_copy(ones_v, deg_sh.at[idx_d], add=True)
        return c

    lax.fori_loop(0, NCH, step, 0)

    plsc.subcore_barrier()

    pltpu.sync_copy(deg_sh.at[pl.ds(base, STR)],
                    deg_out.at[cid, pl.ds(base, STR)])

    @pl.when(sid == NS - 1)
    def _():
        pltpu.sync_copy(deg_sh.at[pl.ds(NS * STR, TAIL)],
                        deg_out.at[cid, pl.ds(NS * STR, TAIL)])


_sc_deg = pl.kernel(
    _sc_deg_body,
    out_type=jax.ShapeDtypeStruct((NC, N, H), jnp.float32),
    mesh=_mesh,
    scratch_types=[
        pltpu.VMEM((C,), jnp.int32),
        pltpu.VMEM((C, H), jnp.float32),
        pltpu.VMEM_SHARED((AGG_ROWS, H), jnp.float32),
    ],
)


def _sc_agg_body(h_hbm, eidx_hbm, agg_out,
                 idx_a, idx_b, rows_a, rows_b,
                 agg_sh, sem_a, sem_b, sem_ia, sem_ib):
    cid = lax.axis_index("c")
    sid = lax.axis_index("s")
    wid = cid * NS + sid
    base = sid * STR

    def zrow(r, c):
        for k in range(H // 16):
            rows_a[r, pl.ds(k * 16, 16)] = _zero_vec()
        return c

    lax.fori_loop(0, C, zrow, 0)

    for j in range(STR // C):
        pltpu.sync_copy(rows_a, agg_sh.at[pl.ds(base + j * C, C)])
    rem = STR - (STR // C) * C
    if rem:
        pltpu.sync_copy(rows_a.at[pl.ds(0, rem)],
                        agg_sh.at[pl.ds(base + STR - rem, rem)])

    @pl.when(sid == NS - 1)
    def _():
        pltpu.sync_copy(rows_a.at[pl.ds(0, TAIL)],
                        agg_sh.at[pl.ds(NS * STR, TAIL)])

    plsc.subcore_barrier()

    pltpu.sync_copy(eidx_hbm.at[wid, 0], idx_a)
    pltpu.async_copy(h_hbm.at[idx_a.at[0]], rows_a, sem_a)
    pltpu.async_copy(eidx_hbm.at[wid, 1], idx_b, sem_ib)

    def pair(jp, c):
        j0 = jp * 2
        not_last = jp < NCH // 2 - 1
        pltpu.make_async_copy(h_hbm.at[idx_a.at[0]], rows_a, sem_a).wait()
        pltpu.make_async_copy(eidx_hbm.at[wid, j0 + 1], idx_b, sem_ib).wait()
        pltpu.async_copy(h_hbm.at[idx_b.at[0]], rows_b, sem_b)
        pltpu.sync_copy(rows_a, agg_sh.at[idx_a.at[1]], add=True)

        @pl.when(not_last)
        def _():
            pltpu.async_copy(eidx_hbm.at[wid, j0 + 2], idx_a, sem_ia)

        pltpu.make_async_copy(h_hbm.at[idx_b.at[0]], rows_b, sem_b).wait()

        @pl.when(not_last)
        def _():
            pltpu.make_async_copy(eidx_hbm.at[wid, j0 + 2], idx_a,
                                  sem_ia).wait()
            pltpu.async_copy(h_hbm.at[idx_a.at[0]], rows_a, sem_a)

        pltpu.sync_copy(rows_b, agg_sh.at[idx_b.at[1]], add=True)

        @pl.when(not_last)
        def _():
            pltpu.async_copy(eidx_hbm.at[wid, j0 + 3], idx_b, sem_ib)

        return c

    lax.fori_loop(0, NCH // 2, pair, 0)

    plsc.subcore_barrier()

    pltpu.sync_copy(agg_sh.at[pl.ds(base, STR)],
                    agg_out.at[cid, pl.ds(base, STR)])

    @pl.when(sid == NS - 1)
    def _():
        pltpu.sync_copy(agg_sh.at[pl.ds(NS * STR, TAIL)],
                        agg_out.at[cid, pl.ds(NS * STR, TAIL)])


_sc_agg = pl.kernel(
    _sc_agg_body,
    out_type=jax.ShapeDtypeStruct((NC, N, H), jnp.float32),
    mesh=_mesh,
    scratch_types=[
        pltpu.VMEM((2, C), jnp.int32),
        pltpu.VMEM((2, C), jnp.int32),
        pltpu.VMEM((C, H), jnp.float32),
        pltpu.VMEM((C, H), jnp.float32),
        pltpu.VMEM_SHARED((AGG_ROWS, H), jnp.float32),
        pltpu.SemaphoreType.DMA,
        pltpu.SemaphoreType.DMA,
        pltpu.SemaphoreType.DMA,
        pltpu.SemaphoreType.DMA,
    ],
)


def _encode_body(x_ref, w_ref, b_ref, d_ref, o_ref, dinv_ref):
    o_ref[...] = jnp.maximum(
        jnp.dot(x_ref[...], w_ref[...], preferred_element_type=jnp.float32)
        + b_ref[...], 0.0)
    dinv = 1.0 / jnp.maximum(d_ref[0, :, 0:1] + d_ref[1, :, 0:1], 1.0)
    dinv_ref[...] = jnp.broadcast_to(dinv, dinv_ref.shape)


_encode = pl.pallas_call(
    _encode_body,
    grid=(GRID,),
    in_specs=[
        pl.BlockSpec((ROWS_BLK, D), lambda i: (i, 0)),
        pl.BlockSpec((D, H), lambda i: (0, 0)),
        pl.BlockSpec((1, H), lambda i: (0, 0)),
        pl.BlockSpec((NC, ROWS_BLK, H), lambda i: (0, i, 0)),
    ],
    out_specs=[
        pl.BlockSpec((ROWS_BLK, H), lambda i: (i, 0)),
        pl.BlockSpec((ROWS_BLK, DEGW), lambda i: (i, 0)),
    ],
    out_shape=[
        jax.ShapeDtypeStruct((N, H), jnp.float32),
        jax.ShapeDtypeStruct((N, DEGW), jnp.float32),
    ],
)


def _update_body(h_ref, a_ref, d_ref, ws_ref, wn_ref, b_ref, o_ref):
    agg = (a_ref[0] + a_ref[1]) * d_ref[:, 0:1]
    o_ref[...] = jnp.maximum(
        jnp.dot(h_ref[...], ws_ref[...], preferred_element_type=jnp.float32)
        + jnp.dot(agg, wn_ref[...], preferred_element_type=jnp.float32)
        + b_ref[...], 0.0)


_update = pl.pallas_call(
    _update_body,
    grid=(GRID,),
    in_specs=[
        pl.BlockSpec((ROWS_BLK, H), lambda i: (i, 0)),
        pl.BlockSpec((NC, ROWS_BLK, H), lambda i: (0, i, 0)),
        pl.BlockSpec((ROWS_BLK, DEGW), lambda i: (i, 0)),
        pl.BlockSpec((H, H), lambda i: (0, 0)),
        pl.BlockSpec((H, H), lambda i: (0, 0)),
        pl.BlockSpec((1, H), lambda i: (0, 0)),
    ],
    out_specs=pl.BlockSpec((ROWS_BLK, H), lambda i: (i, 0)),
    out_shape=jax.ShapeDtypeStruct((N, H), jnp.float32),
)


def _final_body(h_ref, a_ref, d_ref, ws_ref, wn_ref, b_ref,
                wp1_ref, bp1_ref, wp2_ref, bp2_ref, o_ref, acc_ref):
    i = pl.program_id(0)
    agg = (a_ref[0] + a_ref[1]) * d_ref[:, 0:1]
    h3 = jnp.maximum(
        jnp.dot(h_ref[...], ws_ref[...], preferred_element_type=jnp.float32)
        + jnp.dot(agg, wn_ref[...], preferred_element_type=jnp.float32)
        + b_ref[...], 0.0)

    @pl.when(i == 0)
    def _():
        acc_ref[...] = jnp.zeros_like(acc_ref)

    acc_ref[...] += jnp.sum(h3, axis=0, keepdims=True)

    @pl.when(i == GRID - 1)
    def _():
        g = acc_ref[...] * (1.0 / N)
        p = jnp.maximum(
            jnp.dot(g, wp1_ref[...], preferred_element_type=jnp.float32)
            + bp1_ref[...], 0.0)
        o_ref[...] = (jnp.dot(p, wp2_ref[...],
                              preferred_element_type=jnp.float32)
                      + bp2_ref[...])


_final = pl.pallas_call(
    _final_body,
    grid=(GRID,),
    in_specs=[
        pl.BlockSpec((ROWS_BLK, H), lambda i: (i, 0)),
        pl.BlockSpec((NC, ROWS_BLK, H), lambda i: (0, i, 0)),
        pl.BlockSpec((ROWS_BLK, DEGW), lambda i: (i, 0)),
        pl.BlockSpec((H, H), lambda i: (0, 0)),
        pl.BlockSpec((H, H), lambda i: (0, 0)),
        pl.BlockSpec((1, H), lambda i: (0, 0)),
        pl.BlockSpec((H, OUT), lambda i: (0, 0)),
        pl.BlockSpec((1, OUT), lambda i: (0, 0)),
        pl.BlockSpec((OUT, OUT), lambda i: (0, 0)),
        pl.BlockSpec((1, OUT), lambda i: (0, 0)),
    ],
    out_specs=pl.BlockSpec((1, OUT), lambda i: (0, 0)),
    out_shape=jax.ShapeDtypeStruct((1, OUT), jnp.float32),
    scratch_shapes=[pltpu.VMEM((1, H), jnp.float32)],
)


def kernel(x, edge_index, W_enc, b_enc, Ws0, Wn0, bb0, Ws1, Wn1, bb1,
           Ws2, Wn2, bb2, Wp1, bp1, Wp2, bp2):
    pad = EP - EW
    src = jnp.pad(edge_index[0].reshape(NW, EW), ((0, 0), (0, pad)),
                  constant_values=0).reshape(NW, NCH, C)
    dst = jnp.pad(edge_index[1].reshape(NW, EW), ((0, 0), (0, pad)),
                  constant_values=TRASH).reshape(NW, NCH, C)
    eidx = jnp.stack([src, dst], axis=2)

    deg2 = _sc_deg(dst)
    h, dinv = _encode(x, W_enc, b_enc.reshape(1, H), deg2)
    agg = _sc_agg(h, eidx)
    h = _update(h, agg, dinv, Ws0, Wn0, bb0.reshape(1, H))
    agg = _sc_agg(h, eidx)
    h = _update(h, agg, dinv, Ws1, Wn1, bb1.reshape(1, H))
    agg = _sc_agg(h, eidx)
    out = _final(h, agg, dinv, Ws2, Wn2, bb2.reshape(1, H),
                 Wp1, bp1.reshape(1, OUT), Wp2, bp2.reshape(1, OUT))
    return out.reshape(OUT)


# speedup vs baseline: 3.9550x; 1.0479x over previous
import jax
import jax.numpy as jnp
from jax import lax
from jax.experimental import pallas as pl
from jax.experimental.pallas import tpu as pltpu
from jax.experimental.pallas import tpu_sc as plsc

N = 10000
D = 128
H = 128
OUT = 256
E = 320000

NC = 2
NS = 16
NW = NC * NS
EW = E // NW
C = 80
NCH = 128
GD = 4
ID = 8
EP = NCH * C
TRASH = N
AGG_ROWS = N + 8
STR = 624
TAIL = N - NS * STR
DEGW = 16

ROWS_BLK = 1000
GRID = N // ROWS_BLK

_mesh = plsc.VectorSubcoreMesh(core_axis_name="c", subcore_axis_name="s",
                               num_cores=NC, num_subcores=NS)


def _zero_vec():
    return jnp.zeros((16,), jnp.float32)


def _sc_deg_body(dst_hbm, deg_out, idx_d, ones_v, deg_sh):
    cid = lax.axis_index("c")
    sid = lax.axis_index("s")
    wid = cid * NS + sid
    base = sid * STR

    def zrow(r, c):
        for k in range(H // 16):
            ones_v[r, pl.ds(k * 16, 16)] = _zero_vec()
        return c

    lax.fori_loop(0, C, zrow, 0)

    for j in range(STR // C):
        pltpu.sync_copy(ones_v, deg_sh.at[pl.ds(base + j * C, C)])
    rem = STR - (STR // C) * C
    if rem:
        pltpu.sync_copy(ones_v.at[pl.ds(0, rem)],
                        deg_sh.at[pl.ds(base + STR - rem, rem)])

    @pl.when(sid == NS - 1)
    def _():
        pltpu.sync_copy(ones_v.at[pl.ds(0, TAIL)],
                        deg_sh.at[pl.ds(NS * STR, TAIL)])

    def onesrow(r, c):
        for k in range(H // 16):
            ones_v[r, pl.ds(k * 16, 16)] = jnp.ones((16,), jnp.float32)
        return c

    lax.fori_loop(0, C, onesrow, 0)

    plsc.subcore_barrier()

    def step(j, c):
        pltpu.sync_copy(dst_hbm.at[wid, j], idx_d)
        pltpu.sync_copy(ones_v, deg_sh.at[idx_d], add=True)
        return c

    lax.fori_loop(0, NCH, step, 0)

    plsc.subcore_barrier()

    pltpu.sync_copy(deg_sh.at[pl.ds(base, STR)],
                    deg_out.at[cid, pl.ds(base, STR)])

    @pl.when(sid == NS - 1)
    def _():
        pltpu.sync_copy(deg_sh.at[pl.ds(NS * STR, TAIL)],
                        deg_out.at[cid, pl.ds(NS * STR, TAIL)])


_sc_deg = pl.kernel(
    _sc_deg_body,
    out_type=jax.ShapeDtypeStruct((NC, N, H), jnp.float32),
    mesh=_mesh,
    scratch_types=[
        pltpu.VMEM((C,), jnp.int32),
        pltpu.VMEM((C, H), jnp.float32),
        pltpu.VMEM_SHARED((AGG_ROWS, H), jnp.float32),
    ],
)


def _sc_agg_body(h_hbm, eidx_hbm, agg_out, *rest):
    ib = rest[:ID]
    rows = rest[ID:ID + GD]
    agg_sh = rest[ID + GD]
    sg = rest[ID + GD + 1:ID + GD + 1 + GD]
    si = rest[ID + GD + 1 + GD:]
    cid = lax.axis_index("c")
    sid = lax.axis_index("s")
    wid = cid * NS + sid
    base = sid * STR

    def zrow(r, c):
        for k in range(H // 16):
            rows[0][r, pl.ds(k * 16, 16)] = _zero_vec()
        return c

    lax.fori_loop(0, C, zrow, 0)

    for j in range(STR // C):
        pltpu.sync_copy(rows[0], agg_sh.at[pl.ds(base + j * C, C)])
    rem = STR - (STR // C) * C
    if rem:
        pltpu.sync_copy(rows[0].at[pl.ds(0, rem)],
                        agg_sh.at[pl.ds(base + STR - rem, rem)])

    @pl.when(sid == NS - 1)
    def _():
        pltpu.sync_copy(rows[0].at[pl.ds(0, TAIL)],
                        agg_sh.at[pl.ds(NS * STR, TAIL)])

    plsc.subcore_barrier()

    for k in range(GD):
        pltpu.sync_copy(eidx_hbm.at[wid, k], ib[k])
    for k in range(GD):
        pltpu.async_copy(h_hbm.at[ib[k].at[0]], rows[k], sg[k])
    for k in range(GD, ID):
        pltpu.async_copy(eidx_hbm.at[wid, k], ib[k], si[k])

    def block(jo, c):
        for k in range(ID):
            j = jo * ID + k
            r = k % GD
            pltpu.make_async_copy(h_hbm.at[ib[k].at[0]], rows[r],
                                  sg[r]).wait()
            pltpu.sync_copy(rows[r], agg_sh.at[ib[k].at[1]], add=True)

            @pl.when(j < NCH - ID)
            def _():
                pltpu.async_copy(eidx_hbm.at[wid, j + ID], ib[k], si[k])

            @pl.when(j < NCH - GD)
            def _():
                kn = (k + GD) % ID
                pltpu.make_async_copy(eidx_hbm.at[wid, 0], ib[kn],
                                      si[kn]).wait()
                pltpu.async_copy(h_hbm.at[ib[kn].at[0]], rows[r], sg[r])

        return c

    lax.fori_loop(0, NCH // ID, block, 0)

    plsc.subcore_barrier()

    pltpu.sync_copy(agg_sh.at[pl.ds(base, STR)],
                    agg_out.at[cid, pl.ds(base, STR)])

    @pl.when(sid == NS - 1)
    def _():
        pltpu.sync_copy(agg_sh.at[pl.ds(NS * STR, TAIL)],
                        agg_out.at[cid, pl.ds(NS * STR, TAIL)])


_sc_agg = pl.kernel(
    _sc_agg_body,
    out_type=jax.ShapeDtypeStruct((NC, N, H), jnp.float32),
    mesh=_mesh,
    scratch_types=(
        [pltpu.VMEM((2, C), jnp.int32) for _ in range(ID)]
        + [pltpu.VMEM((C, H), jnp.float32) for _ in range(GD)]
        + [pltpu.VMEM_SHARED((AGG_ROWS, H), jnp.float32)]
        + [pltpu.SemaphoreType.DMA for _ in range(GD + ID)]
    ),
)


def _encode_body(x_ref, w_ref, b_ref, d_ref, o_ref, dinv_ref):
    o_ref[...] = jnp.maximum(
        jnp.dot(x_ref[...], w_ref[...], preferred_element_type=jnp.float32)
        + b_ref[...], 0.0)
    dinv = 1.0 / jnp.maximum(d_ref[0, :, 0:1] + d_ref[1, :, 0:1], 1.0)
    dinv_ref[...] = jnp.broadcast_to(dinv, dinv_ref.shape)


_encode = pl.pallas_call(
    _encode_body,
    grid=(GRID,),
    in_specs=[
        pl.BlockSpec((ROWS_BLK, D), lambda i: (i, 0)),
        pl.BlockSpec((D, H), lambda i: (0, 0)),
        pl.BlockSpec((1, H), lambda i: (0, 0)),
        pl.BlockSpec((NC, ROWS_BLK, H), lambda i: (0, i, 0)),
    ],
    out_specs=[
        pl.BlockSpec((ROWS_BLK, H), lambda i: (i, 0)),
        pl.BlockSpec((ROWS_BLK, DEGW), lambda i: (i, 0)),
    ],
    out_shape=[
        jax.ShapeDtypeStruct((N, H), jnp.float32),
        jax.ShapeDtypeStruct((N, DEGW), jnp.float32),
    ],
)


def _update_body(h_ref, a_ref, d_ref, ws_ref, wn_ref, b_ref, o_ref):
    agg = (a_ref[0] + a_ref[1]) * d_ref[:, 0:1]
    o_ref[...] = jnp.maximum(
        jnp.dot(h_ref[...], ws_ref[...], preferred_element_type=jnp.float32)
        + jnp.dot(agg, wn_ref[...], preferred_element_type=jnp.float32)
        + b_ref[...], 0.0)


_update = pl.pallas_call(
    _update_body,
    grid=(GRID,),
    in_specs=[
        pl.BlockSpec((ROWS_BLK, H), lambda i: (i, 0)),
        pl.BlockSpec((NC, ROWS_BLK, H), lambda i: (0, i, 0)),
        pl.BlockSpec((ROWS_BLK, DEGW), lambda i: (i, 0)),
        pl.BlockSpec((H, H), lambda i: (0, 0)),
        pl.BlockSpec((H, H), lambda i: (0, 0)),
        pl.BlockSpec((1, H), lambda i: (0, 0)),
    ],
    out_specs=pl.BlockSpec((ROWS_BLK, H), lambda i: (i, 0)),
    out_shape=jax.ShapeDtypeStruct((N, H), jnp.float32),
)


def _final_body(h_ref, a_ref, d_ref, ws_ref, wn_ref, b_ref,
                wp1_ref, bp1_ref, wp2_ref, bp2_ref, o_ref, acc_ref):
    i = pl.program_id(0)
    agg = (a_ref[0] + a_ref[1]) * d_ref[:, 0:1]
    h3 = jnp.maximum(
        jnp.dot(h_ref[...], ws_ref[...], preferred_element_type=jnp.float32)
        + jnp.dot(agg, wn_ref[...], preferred_element_type=jnp.float32)
        + b_ref[...], 0.0)

    @pl.when(i == 0)
    def _():
        acc_ref[...] = jnp.zeros_like(acc_ref)

    acc_ref[...] += jnp.sum(h3, axis=0, keepdims=True)

    @pl.when(i == GRID - 1)
    def _():
        g = acc_ref[...] * (1.0 / N)
        p = jnp.maximum(
            jnp.dot(g, wp1_ref[...], preferred_element_type=jnp.float32)
            + bp1_ref[...], 0.0)
        o_ref[...] = (jnp.dot(p, wp2_ref[...],
                              preferred_element_type=jnp.float32)
                      + bp2_ref[...])


_final = pl.pallas_call(
    _final_body,
    grid=(GRID,),
    in_specs=[
        pl.BlockSpec((ROWS_BLK, H), lambda i: (i, 0)),
        pl.BlockSpec((NC, ROWS_BLK, H), lambda i: (0, i, 0)),
        pl.BlockSpec((ROWS_BLK, DEGW), lambda i: (i, 0)),
        pl.BlockSpec((H, H), lambda i: (0, 0)),
        pl.BlockSpec((H, H), lambda i: (0, 0)),
        pl.BlockSpec((1, H), lambda i: (0, 0)),
        pl.BlockSpec((H, OUT), lambda i: (0, 0)),
        pl.BlockSpec((1, OUT), lambda i: (0, 0)),
        pl.BlockSpec((OUT, OUT), lambda i: (0, 0)),
        pl.BlockSpec((1, OUT), lambda i: (0, 0)),
    ],
    out_specs=pl.BlockSpec((1, OUT), lambda i: (0, 0)),
    out_shape=jax.ShapeDtypeStruct((1, OUT), jnp.float32),
    scratch_shapes=[pltpu.VMEM((1, H), jnp.float32)],
)


def kernel(x, edge_index, W_enc, b_enc, Ws0, Wn0, bb0, Ws1, Wn1, bb1,
           Ws2, Wn2, bb2, Wp1, bp1, Wp2, bp2):
    pad = EP - EW
    src = jnp.pad(edge_index[0].reshape(NW, EW), ((0, 0), (0, pad)),
                  constant_values=0).reshape(NW, NCH, C)
    dst = jnp.pad(edge_index[1].reshape(NW, EW), ((0, 0), (0, pad)),
                  constant_values=TRASH).reshape(NW, NCH, C)
    eidx = jnp.stack([src, dst], axis=2)

    deg2 = _sc_deg(dst)
    h, dinv = _encode(x, W_enc, b_enc.reshape(1, H), deg2)
    agg = _sc_agg(h, eidx)
    h = _update(h, agg, dinv, Ws0, Wn0, bb0.reshape(1, H))
    agg = _sc_agg(h, eidx)
    h = _update(h, agg, dinv, Ws1, Wn1, bb1.reshape(1, H))
    agg = _sc_agg(h, eidx)
    out = _final(h, agg, dinv, Ws2, Wn2, bb2.reshape(1, H),
                 Wp1, bp1.reshape(1, OUT), Wp2, bp2.reshape(1, OUT))
    return out.reshape(OUT)


# speedup vs baseline: 4.1058x; 1.0381x over previous
import jax
import jax.numpy as jnp
from jax import lax
from jax.experimental import pallas as pl
from jax.experimental.pallas import tpu as pltpu
from jax.experimental.pallas import tpu_sc as plsc

N = 10000
D = 128
H = 128
OUT = 256
E = 320000

NC = 2
NS = 16
NW = NC * NS
EW = E // NW
C = 80
NCH = 128
GD = 4
ID = 8
EP = NCH * C
TRASH = N
AGG_ROWS = N + 8
STR = 624
TAIL = N - NS * STR
DEGW = 16

ROWS_BLK = 1000
GRID = N // ROWS_BLK

_mesh = plsc.VectorSubcoreMesh(core_axis_name="c", subcore_axis_name="s",
                               num_cores=NC, num_subcores=NS)


def _zero_vec():
    return jnp.zeros((16,), jnp.float32)


def _sc_deg_body(eidx_hbm, deg_out, *rest):
    ib = rest[:ID]
    ones_v = rest[ID]
    deg_sh = rest[ID + 1]
    si = rest[ID + 2:]
    cid = lax.axis_index("c")
    sid = lax.axis_index("s")
    wid = cid * NS + sid
    base = sid * STR

    def zrow(r, c):
        for k in range(H // 16):
            ones_v[r, pl.ds(k * 16, 16)] = _zero_vec()
        return c

    lax.fori_loop(0, C, zrow, 0)

    for j in range(STR // C):
        pltpu.sync_copy(ones_v, deg_sh.at[pl.ds(base + j * C, C)])
    rem = STR - (STR // C) * C
    if rem:
        pltpu.sync_copy(ones_v.at[pl.ds(0, rem)],
                        deg_sh.at[pl.ds(base + STR - rem, rem)])

    @pl.when(sid == NS - 1)
    def _():
        pltpu.sync_copy(ones_v.at[pl.ds(0, TAIL)],
                        deg_sh.at[pl.ds(NS * STR, TAIL)])

    def onesrow(r, c):
        for k in range(H // 16):
            ones_v[r, pl.ds(k * 16, 16)] = jnp.ones((16,), jnp.float32)
        return c

    lax.fori_loop(0, C, onesrow, 0)

    plsc.subcore_barrier()

    for k in range(ID):
        pltpu.async_copy(eidx_hbm.at[wid, k], ib[k], si[k])

    def block(jo, c):
        for k in range(ID):
            j = jo * ID + k
            pltpu.make_async_copy(eidx_hbm.at[wid, 0], ib[k], si[k]).wait()
            pltpu.sync_copy(ones_v, deg_sh.at[ib[k].at[1]], add=True)

            @pl.when(j < NCH - ID)
            def _():
                pltpu.async_copy(eidx_hbm.at[wid, j + ID], ib[k], si[k])

        return c

    lax.fori_loop(0, NCH // ID, block, 0)

    plsc.subcore_barrier()

    pltpu.sync_copy(deg_sh.at[pl.ds(base, STR)],
                    deg_out.at[cid, pl.ds(base, STR)])

    @pl.when(sid == NS - 1)
    def _():
        pltpu.sync_copy(deg_sh.at[pl.ds(NS * STR, TAIL)],
                        deg_out.at[cid, pl.ds(NS * STR, TAIL)])


_sc_deg = pl.kernel(
    _sc_deg_body,
    out_type=jax.ShapeDtypeStruct((NC, N, H), jnp.float32),
    mesh=_mesh,
    scratch_types=(
        [pltpu.VMEM((2, C), jnp.int32) for _ in range(ID)]
        + [pltpu.VMEM((C, H), jnp.float32)]
        + [pltpu.VMEM_SHARED((AGG_ROWS, H), jnp.float32)]
        + [pltpu.SemaphoreType.DMA for _ in range(ID)]
    ),
)


def _sc_agg_body(h_hbm, eidx_hbm, agg_out, *rest):
    ib = rest[:ID]
    rows = rest[ID:ID + GD]
    agg_sh = rest[ID + GD]
    sg = rest[ID + GD + 1:ID + GD + 1 + GD]
    si = rest[ID + GD + 1 + GD:]
    cid = lax.axis_index("c")
    sid = lax.axis_index("s")
    wid = cid * NS + sid
    base = sid * STR

    def zrow(r, c):
        for k in range(H // 16):
            rows[0][r, pl.ds(k * 16, 16)] = _zero_vec()
        return c

    lax.fori_loop(0, C, zrow, 0)

    for j in range(STR // C):
        pltpu.sync_copy(rows[0], agg_sh.at[pl.ds(base + j * C, C)])
    rem = STR - (STR // C) * C
    if rem:
        pltpu.sync_copy(rows[0].at[pl.ds(0, rem)],
                        agg_sh.at[pl.ds(base + STR - rem, rem)])

    @pl.when(sid == NS - 1)
    def _():
        pltpu.sync_copy(rows[0].at[pl.ds(0, TAIL)],
                        agg_sh.at[pl.ds(NS * STR, TAIL)])

    plsc.subcore_barrier()

    for k in range(GD):
        pltpu.sync_copy(eidx_hbm.at[wid, k], ib[k])
    for k in range(GD):
        pltpu.async_copy(h_hbm.at[ib[k].at[0]], rows[k], sg[k])
    for k in range(GD, ID):
        pltpu.async_copy(eidx_hbm.at[wid, k], ib[k], si[k])

    def block(jo, c):
        for k in range(ID):
            j = jo * ID + k
            r = k % GD
            pltpu.make_async_copy(h_hbm.at[ib[k].at[0]], rows[r],
                                  sg[r]).wait()
            pltpu.sync_copy(rows[r], agg_sh.at[ib[k].at[1]], add=True)

            @pl.when(j < NCH - ID)
            def _():
                pltpu.async_copy(eidx_hbm.at[wid, j + ID], ib[k], si[k])

            @pl.when(j < NCH - GD)
            def _():
                kn = (k + GD) % ID
                pltpu.make_async_copy(eidx_hbm.at[wid, 0], ib[kn],
                                      si[kn]).wait()
                pltpu.async_copy(h_hbm.at[ib[kn].at[0]], rows[r], sg[r])

        return c

    lax.fori_loop(0, NCH // ID, block, 0)

    plsc.subcore_barrier()

    pltpu.sync_copy(agg_sh.at[pl.ds(base, STR)],
                    agg_out.at[cid, pl.ds(base, STR)])

    @pl.when(sid == NS - 1)
    def _():
        pltpu.sync_copy(agg_sh.at[pl.ds(NS * STR, TAIL)],
                        agg_out.at[cid, pl.ds(NS * STR, TAIL)])


_sc_agg = pl.kernel(
    _sc_agg_body,
    out_type=jax.ShapeDtypeStruct((NC, N, H), jnp.float32),
    mesh=_mesh,
    scratch_types=(
        [pltpu.VMEM((2, C), jnp.int32) for _ in range(ID)]
        + [pltpu.VMEM((C, H), jnp.float32) for _ in range(GD)]
        + [pltpu.VMEM_SHARED((AGG_ROWS, H), jnp.float32)]
        + [pltpu.SemaphoreType.DMA for _ in range(GD + ID)]
    ),
)


def _encode_body(x_ref, w_ref, b_ref, d_ref, o_ref, dinv_ref):
    o_ref[...] = jnp.maximum(
        jnp.dot(x_ref[...], w_ref[...], preferred_element_type=jnp.float32)
        + b_ref[...], 0.0)
    dinv = 1.0 / jnp.maximum(d_ref[0, :, 0:1] + d_ref[1, :, 0:1], 1.0)
    dinv_ref[...] = jnp.broadcast_to(dinv, dinv_ref.shape)


_encode = pl.pallas_call(
    _encode_body,
    grid=(GRID,),
    in_specs=[
        pl.BlockSpec((ROWS_BLK, D), lambda i: (i, 0)),
        pl.BlockSpec((D, H), lambda i: (0, 0)),
        pl.BlockSpec((1, H), lambda i: (0, 0)),
        pl.BlockSpec((NC, ROWS_BLK, H), lambda i: (0, i, 0)),
    ],
    out_specs=[
        pl.BlockSpec((ROWS_BLK, H), lambda i: (i, 0)),
        pl.BlockSpec((ROWS_BLK, DEGW), lambda i: (i, 0)),
    ],
    out_shape=[
        jax.ShapeDtypeStruct((N, H), jnp.float32),
        jax.ShapeDtypeStruct((N, DEGW), jnp.float32),
    ],
)


def _update_body(h_ref, a_ref, d_ref, ws_ref, wn_ref, b_ref, o_ref):
    agg = (a_ref[0] + a_ref[1]) * d_ref[:, 0:1]
    o_ref[...] = jnp.maximum(
        jnp.dot(h_ref[...], ws_ref[...], preferred_element_type=jnp.float32)
        + jnp.dot(agg, wn_ref[...], preferred_element_type=jnp.float32)
        + b_ref[...], 0.0)


_update = pl.pallas_call(
    _update_body,
    grid=(GRID,),
    in_specs=[
        pl.BlockSpec((ROWS_BLK, H), lambda i: (i, 0)),
        pl.BlockSpec((NC, ROWS_BLK, H), lambda i: (0, i, 0)),
        pl.BlockSpec((ROWS_BLK, DEGW), lambda i: (i, 0)),
        pl.BlockSpec((H, H), lambda i: (0, 0)),
        pl.BlockSpec((H, H), lambda i: (0, 0)),
        pl.BlockSpec((1, H), lambda i: (0, 0)),
    ],
    out_specs=pl.BlockSpec((ROWS_BLK, H), lambda i: (i, 0)),
    out_shape=jax.ShapeDtypeStruct((N, H), jnp.float32),
)


def _final_body(h_ref, a_ref, d_ref, ws_ref, wn_ref, b_ref,
                wp1_ref, bp1_ref, wp2_ref, bp2_ref, o_ref, acc_ref):
    i = pl.program_id(0)
    agg = (a_ref[0] + a_ref[1]) * d_ref[:, 0:1]
    h3 = jnp.maximum(
        jnp.dot(h_ref[...], ws_ref[...], preferred_element_type=jnp.float32)
        + jnp.dot(agg, wn_ref[...], preferred_element_type=jnp.float32)
        + b_ref[...], 0.0)

    @pl.when(i == 0)
    def _():
        acc_ref[...] = jnp.zeros_like(acc_ref)

    acc_ref[...] += jnp.sum(h3, axis=0, keepdims=True)

    @pl.when(i == GRID - 1)
    def _():
        g = acc_ref[...] * (1.0 / N)
        p = jnp.maximum(
            jnp.dot(g, wp1_ref[...], preferred_element_type=jnp.float32)
            + bp1_ref[...], 0.0)
        o_ref[...] = (jnp.dot(p, wp2_ref[...],
                              preferred_element_type=jnp.float32)
                      + bp2_ref[...])


_final = pl.pallas_call(
    _final_body,
    grid=(GRID,),
    in_specs=[
        pl.BlockSpec((ROWS_BLK, H), lambda i: (i, 0)),
        pl.BlockSpec((NC, ROWS_BLK, H), lambda i: (0, i, 0)),
        pl.BlockSpec((ROWS_BLK, DEGW), lambda i: (i, 0)),
        pl.BlockSpec((H, H), lambda i: (0, 0)),
        pl.BlockSpec((H, H), lambda i: (0, 0)),
        pl.BlockSpec((1, H), lambda i: (0, 0)),
        pl.BlockSpec((H, OUT), lambda i: (0, 0)),
        pl.BlockSpec((1, OUT), lambda i: (0, 0)),
        pl.BlockSpec((OUT, OUT), lambda i: (0, 0)),
        pl.BlockSpec((1, OUT), lambda i: (0, 0)),
    ],
    out_specs=pl.BlockSpec((1, OUT), lambda i: (0, 0)),
    out_shape=jax.ShapeDtypeStruct((1, OUT), jnp.float32),
    scratch_shapes=[pltpu.VMEM((1, H), jnp.float32)],
)


def kernel(x, edge_index, W_enc, b_enc, Ws0, Wn0, bb0, Ws1, Wn1, bb1,
           Ws2, Wn2, bb2, Wp1, bp1, Wp2, bp2):
    pad = EP - EW
    src = jnp.pad(edge_index[0].reshape(NW, EW), ((0, 0), (0, pad)),
                  constant_values=0).reshape(NW, NCH, C)
    dst = jnp.pad(edge_index[1].reshape(NW, EW), ((0, 0), (0, pad)),
                  constant_values=TRASH).reshape(NW, NCH, C)
    eidx = jnp.stack([src, dst], axis=2)

    deg2 = _sc_deg(eidx)
    h, dinv = _encode(x, W_enc, b_enc.reshape(1, H), deg2)
    agg = _sc_agg(h, eidx)
    h = _update(h, agg, dinv, Ws0, Wn0, bb0.reshape(1, H))
    agg = _sc_agg(h, eidx)
    h = _update(h, agg, dinv, Ws1, Wn1, bb1.reshape(1, H))
    agg = _sc_agg(h, eidx)
    out = _final(h, agg, dinv, Ws2, Wn2, bb2.reshape(1, H),
                 Wp1, bp1.reshape(1, OUT), Wp2, bp2.reshape(1, OUT))
    return out.reshape(OUT)


# speedup vs baseline: 7.4411x; 1.8123x over previous
import jax
import jax.numpy as jnp
from jax import lax
from jax.experimental import pallas as pl
from jax.experimental.pallas import tpu as pltpu
from jax.experimental.pallas import tpu_sc as plsc

N = 10000
D = 128
H = 128
OUT = 256
E = 320000

NC = 2
NS = 16
NW = NC * NS
EW = E // NW
C = 112
NCH = 90
GD = 3
ID = 6
EP = NCH * C
TRASH = N
AGG_ROWS = N + 8
STR = 624
TAIL = N - NS * STR
DEGW = 16

ROWS_BLK = 1000
GRID = N // ROWS_BLK

_mesh = plsc.VectorSubcoreMesh(core_axis_name="c", subcore_axis_name="s",
                               num_cores=NC, num_subcores=NS)


def _zero_vec():
    return jnp.zeros((16,), jnp.float32)


def _sc_deg_body(eidx_hbm, deg_out, *rest):
    ib = rest[:ID]
    ones_v = rest[ID]
    deg_sh = rest[ID + 1]
    si = rest[ID + 2:]
    cid = lax.axis_index("c")
    sid = lax.axis_index("s")
    wid = cid * NS + sid
    base = sid * STR

    def zrow(r, c):
        for k in range(H // 16):
            ones_v[r, pl.ds(k * 16, 16)] = _zero_vec()
        return c

    lax.fori_loop(0, C, zrow, 0)

    for j in range(STR // C):
        pltpu.sync_copy(ones_v, deg_sh.at[pl.ds(base + j * C, C)])
    rem = STR - (STR // C) * C
    if rem:
        pltpu.sync_copy(ones_v.at[pl.ds(0, rem)],
                        deg_sh.at[pl.ds(base + STR - rem, rem)])

    @pl.when(sid == NS - 1)
    def _():
        pltpu.sync_copy(ones_v.at[pl.ds(0, TAIL)],
                        deg_sh.at[pl.ds(NS * STR, TAIL)])

    def onesrow(r, c):
        for k in range(H // 16):
            ones_v[r, pl.ds(k * 16, 16)] = jnp.ones((16,), jnp.float32)
        return c

    lax.fori_loop(0, C, onesrow, 0)

    plsc.subcore_barrier()

    for k in range(ID):
        pltpu.async_copy(eidx_hbm.at[wid, k], ib[k], si[k])

    def block(jo, c):
        for k in range(ID):
            j = jo * ID + k
            pltpu.make_async_copy(eidx_hbm.at[wid, 0], ib[k], si[k]).wait()
            pltpu.sync_copy(ones_v, deg_sh.at[ib[k].at[1]], add=True)

            @pl.when(j < NCH - ID)
            def _():
                pltpu.async_copy(eidx_hbm.at[wid, j + ID], ib[k], si[k])

        return c

    lax.fori_loop(0, NCH // ID, block, 0)

    plsc.subcore_barrier()

    pltpu.sync_copy(deg_sh.at[pl.ds(base, STR)],
                    deg_out.at[cid, pl.ds(base, STR)])

    @pl.when(sid == NS - 1)
    def _():
        pltpu.sync_copy(deg_sh.at[pl.ds(NS * STR, TAIL)],
                        deg_out.at[cid, pl.ds(NS * STR, TAIL)])


_sc_deg = pl.kernel(
    _sc_deg_body,
    out_type=jax.ShapeDtypeStruct((NC, N, H), jnp.float32),
    mesh=_mesh,
    scratch_types=(
        [pltpu.VMEM((2, C), jnp.int32) for _ in range(ID)]
        + [pltpu.VMEM((C, H), jnp.float32)]
        + [pltpu.VMEM_SHARED((AGG_ROWS, H), jnp.float32)]
        + [pltpu.SemaphoreType.DMA for _ in range(ID)]
    ),
)


def _sc_agg_body(h_hbm, eidx_hbm, agg_out, *rest):
    ib = rest[:ID]
    rows = rest[ID:ID + GD]
    agg_sh = rest[ID + GD]
    sg = rest[ID + GD + 1:ID + GD + 1 + GD]
    si = rest[ID + GD + 1 + GD:]
    cid = lax.axis_index("c")
    sid = lax.axis_index("s")
    wid = cid * NS + sid
    base = sid * STR

    def zrow(r, c):
        for k in range(H // 16):
            rows[0][r, pl.ds(k * 16, 16)] = _zero_vec()
        return c

    lax.fori_loop(0, C, zrow, 0)

    for j in range(STR // C):
        pltpu.sync_copy(rows[0], agg_sh.at[pl.ds(base + j * C, C)])
    rem = STR - (STR // C) * C
    if rem:
        pltpu.sync_copy(rows[0].at[pl.ds(0, rem)],
                        agg_sh.at[pl.ds(base + STR - rem, rem)])

    @pl.when(sid == NS - 1)
    def _():
        pltpu.sync_copy(rows[0].at[pl.ds(0, TAIL)],
                        agg_sh.at[pl.ds(NS * STR, TAIL)])

    plsc.subcore_barrier()

    for k in range(GD):
        pltpu.sync_copy(eidx_hbm.at[wid, k], ib[k])
    for k in range(GD):
        pltpu.async_copy(h_hbm.at[ib[k].at[0]], rows[k], sg[k])
    for k in range(GD, ID):
        pltpu.async_copy(eidx_hbm.at[wid, k], ib[k], si[k])

    def block(jo, c):
        for k in range(ID):
            j = jo * ID + k
            r = k % GD
            pltpu.make_async_copy(h_hbm.at[ib[k].at[0]], rows[r],
                                  sg[r]).wait()
            pltpu.sync_copy(rows[r], agg_sh.at[ib[k].at[1]], add=True)

            @pl.when(j < NCH - ID)
            def _():
                pltpu.async_copy(eidx_hbm.at[wid, j + ID], ib[k], si[k])

            @pl.when(j < NCH - GD)
            def _():
                kn = (k + GD) % ID
                pltpu.make_async_copy(eidx_hbm.at[wid, 0], ib[kn],
                                      si[kn]).wait()
                pltpu.async_copy(h_hbm.at[ib[kn].at[0]], rows[r], sg[r])

        return c

    lax.fori_loop(0, NCH // ID, block, 0)

    plsc.subcore_barrier()

    pltpu.sync_copy(agg_sh.at[pl.ds(base, STR)],
                    agg_out.at[cid, pl.ds(base, STR)])

    @pl.when(sid == NS - 1)
    def _():
        pltpu.sync_copy(agg_sh.at[pl.ds(NS * STR, TAIL)],
                        agg_out.at[cid, pl.ds(NS * STR, TAIL)])


_sc_agg = pl.kernel(
    _sc_agg_body,
    out_type=jax.ShapeDtypeStruct((NC, N, H), jnp.float32),
    mesh=_mesh,
    scratch_types=(
        [pltpu.VMEM((2, C), jnp.int32) for _ in range(ID)]
        + [pltpu.VMEM((C, H), jnp.float32) for _ in range(GD)]
        + [pltpu.VMEM_SHARED((AGG_ROWS, H), jnp.float32)]
        + [pltpu.SemaphoreType.DMA for _ in range(GD + ID)]
    ),
)


def _encode_body(x_ref, w_ref, b_ref, d_ref, o_ref, dinv_ref):
    o_ref[...] = jnp.maximum(
        jnp.dot(x_ref[...], w_ref[...], preferred_element_type=jnp.float32)
        + b_ref[...], 0.0)
    dinv = 1.0 / jnp.maximum(d_ref[0, :, 0:1] + d_ref[1, :, 0:1], 1.0)
    dinv_ref[...] = jnp.broadcast_to(dinv, dinv_ref.shape)


_encode = pl.pallas_call(
    _encode_body,
    grid=(GRID,),
    in_specs=[
        pl.BlockSpec((ROWS_BLK, D), lambda i: (i, 0)),
        pl.BlockSpec((D, H), lambda i: (0, 0)),
        pl.BlockSpec((1, H), lambda i: (0, 0)),
        pl.BlockSpec((NC, ROWS_BLK, H), lambda i: (0, i, 0)),
    ],
    out_specs=[
        pl.BlockSpec((ROWS_BLK, H), lambda i: (i, 0)),
        pl.BlockSpec((ROWS_BLK, DEGW), lambda i: (i, 0)),
    ],
    out_shape=[
        jax.ShapeDtypeStruct((N, H), jnp.float32),
        jax.ShapeDtypeStruct((N, DEGW), jnp.float32),
    ],
)


def _update_body(h_ref, a_ref, d_ref, ws_ref, wn_ref, b_ref, o_ref):
    agg = (a_ref[0] + a_ref[1]) * d_ref[:, 0:1]
    o_ref[...] = jnp.maximum(
        jnp.dot(h_ref[...], ws_ref[...], preferred_element_type=jnp.float32)
        + jnp.dot(agg, wn_ref[...], preferred_element_type=jnp.float32)
        + b_ref[...], 0.0)


_update = pl.pallas_call(
    _update_body,
    grid=(GRID,),
    in_specs=[
        pl.BlockSpec((ROWS_BLK, H), lambda i: (i, 0)),
        pl.BlockSpec((NC, ROWS_BLK, H), lambda i: (0, i, 0)),
        pl.BlockSpec((ROWS_BLK, DEGW), lambda i: (i, 0)),
        pl.BlockSpec((H, H), lambda i: (0, 0)),
        pl.BlockSpec((H, H), lambda i: (0, 0)),
        pl.BlockSpec((1, H), lambda i: (0, 0)),
    ],
    out_specs=pl.BlockSpec((ROWS_BLK, H), lambda i: (i, 0)),
    out_shape=jax.ShapeDtypeStruct((N, H), jnp.float32),
)


def _final_body(h_ref, a_ref, d_ref, ws_ref, wn_ref, b_ref,
                wp1_ref, bp1_ref, wp2_ref, bp2_ref, o_ref, acc_ref):
    i = pl.program_id(0)
    agg = (a_ref[0] + a_ref[1]) * d_ref[:, 0:1]
    h3 = jnp.maximum(
        jnp.dot(h_ref[...], ws_ref[...], preferred_element_type=jnp.float32)
        + jnp.dot(agg, wn_ref[...], preferred_element_type=jnp.float32)
        + b_ref[...], 0.0)

    @pl.when(i == 0)
    def _():
        acc_ref[...] = jnp.zeros_like(acc_ref)

    acc_ref[...] += jnp.sum(h3, axis=0, keepdims=True)

    @pl.when(i == GRID - 1)
    def _():
        g = acc_ref[...] * (1.0 / N)
        p = jnp.maximum(
            jnp.dot(g, wp1_ref[...], preferred_element_type=jnp.float32)
            + bp1_ref[...], 0.0)
        o_ref[...] = (jnp.dot(p, wp2_ref[...],
                              preferred_element_type=jnp.float32)
                      + bp2_ref[...])


_final = pl.pallas_call(
    _final_body,
    grid=(GRID,),
    in_specs=[
        pl.BlockSpec((ROWS_BLK, H), lambda i: (i, 0)),
        pl.BlockSpec((NC, ROWS_BLK, H), lambda i: (0, i, 0)),
        pl.BlockSpec((ROWS_BLK, DEGW), lambda i: (i, 0)),
        pl.BlockSpec((H, H), lambda i: (0, 0)),
        pl.BlockSpec((H, H), lambda i: (0, 0)),
        pl.BlockSpec((1, H), lambda i: (0, 0)),
        pl.BlockSpec((H, OUT), lambda i: (0, 0)),
        pl.BlockSpec((1, OUT), lambda i: (0, 0)),
        pl.BlockSpec((OUT, OUT), lambda i: (0, 0)),
        pl.BlockSpec((1, OUT), lambda i: (0, 0)),
    ],
    out_specs=pl.BlockSpec((1, OUT), lambda i: (0, 0)),
    out_shape=jax.ShapeDtypeStruct((1, OUT), jnp.float32),
    scratch_shapes=[pltpu.VMEM((1, H), jnp.float32)],
)


def kernel(x, edge_index, W_enc, b_enc, Ws0, Wn0, bb0, Ws1, Wn1, bb1,
           Ws2, Wn2, bb2, Wp1, bp1, Wp2, bp2):
    pad = EP - EW
    src = jnp.pad(edge_index[0].reshape(NW, EW), ((0, 0), (0, pad)),
                  constant_values=0).reshape(NW, NCH, C)
    dst = jnp.pad(edge_index[1].reshape(NW, EW), ((0, 0), (0, pad)),
                  constant_values=TRASH).reshape(NW, NCH, C)
    eidx = jnp.stack([src, dst], axis=2)

    deg2 = _sc_deg(eidx)
    h, dinv = _encode(x, W_enc, b_enc.reshape(1, H), deg2)
    agg = _sc_agg(h, eidx)
    h = _update(h, agg, dinv, Ws0, Wn0, bb0.reshape(1, H))
    agg = _sc_agg(h, eidx)
    h = _update(h, agg, dinv, Ws1, Wn1, bb1.reshape(1, H))
    agg = _sc_agg(h, eidx)
    out = _final(h, agg, dinv, Ws2, Wn2, bb2.reshape(1, H),
                 Wp1, bp1.reshape(1, OUT), Wp2, bp2.reshape(1, OUT))
    return out.reshape(OUT)
